# baseline (speedup 1.0000x reference)
import functools

import jax
import jax.numpy as jnp
from jax import lax
from jax.experimental import pallas as pl
from jax.experimental.pallas import tpu as pltpu
from jax.experimental.pallas import tpu_sc as plsc

HIDDEN = 1024
NUM_CORES = 2
NUM_SUBCORES = 16
COLS = 128
N_SLICES = HIDDEN // (NUM_CORES * COLS)
CHUNK = 64
NBUF = 6
GDEPTH = 2
SDEPTH = NBUF - GDEPTH


def _gather_flat(flat_ids, table):
    B = flat_ids.shape[0]
    V = table.shape[0]
    b_per_w = B // NUM_SUBCORES
    n_per = b_per_w // CHUNK
    rows_per_sub = V // NUM_SUBCORES

    mesh = plsc.VectorSubcoreMesh(core_axis_name="c", subcore_axis_name="s")

    @functools.partial(
        pl.kernel,
        mesh=mesh,
        out_type=jax.ShapeDtypeStruct((B, HIDDEN), jnp.float32),
        scratch_types=[
            pltpu.VMEM_SHARED((V, COLS), jnp.float32),
            pltpu.VMEM((b_per_w,), jnp.int32),
        ]
        + [pltpu.VMEM((CHUNK, COLS), jnp.float32)] * NBUF
        + [pltpu.SemaphoreType.DMA] * NBUF,
    )
    def emb(idx_hbm, table_hbm, out_hbm, shared, idx_v, *rest):
        bufs = rest[:NBUF]
        sems = rest[NBUF:]
        c = lax.axis_index("c")
        s = lax.axis_index("s")
        base = s * b_per_w
        pltpu.sync_copy(idx_hbm.at[pl.ds(base, b_per_w)], idx_v)

        def row_off(G):
            return (G % n_per) * CHUNK

        def col0(G):
            return (c * N_SLICES + G // n_per) * COLS

        def start_gather(G, k):
            pltpu.async_copy(
                shared.at[idx_v.at[pl.ds(row_off(G), CHUNK)]], bufs[k], sems[k]
            )

        def wait_gather(G, k):
            pltpu.make_async_copy(
                shared.at[idx_v.at[pl.ds(row_off(G), CHUNK)]], bufs[k], sems[k]
            ).wait()

        def start_store(G, k):
            pltpu.async_copy(
                bufs[k],
                out_hbm.at[pl.ds(base + row_off(G), CHUNK), pl.ds(col0(G), COLS)],
                sems[k],
            )

        def wait_store(G, k):
            pltpu.make_async_copy(
                bufs[k],
                out_hbm.at[pl.ds(base + row_off(G), CHUNK), pl.ds(col0(G), COLS)],
                sems[k],
            ).wait()

        def phase(G, k, store_wait, gather_start):
            wait_gather(G, k)
            if store_wait:
                wait_store(G - SDEPTH, (k + GDEPTH) % NBUF)
            if gather_start:
                start_gather(G + GDEPTH, (k + GDEPTH) % NBUF)
            start_store(G, k)

        for i in range(N_SLICES):
            g_lo = i * n_per
            g_hi = g_lo + n_per
            if i > 0:
                plsc.subcore_barrier()
            pltpu.sync_copy(
                table_hbm.at[
                    pl.ds(s * rows_per_sub, rows_per_sub),
                    pl.ds((c * N_SLICES + i) * COLS, COLS),
                ],
                shared.at[pl.ds(s * rows_per_sub, rows_per_sub)],
            )
            plsc.subcore_barrier()

            start_gather(g_lo, g_lo % NBUF)
            start_gather(g_lo + 1, (g_lo + 1) % NBUF)
            head = SDEPTH if i == 0 else 0
            for G in range(g_lo, g_lo + head):
                phase(G, G % NBUF, False, True)
            full_lo = g_lo + head
            n_full = (g_hi - GDEPTH) - full_lo
            n_loop = (n_full // NBUF) * NBUF

            def body(j, carry):
                for p in range(NBUF):
                    G = full_lo + j * NBUF + p
                    phase(G, (full_lo + p) % NBUF, True, True)
                return carry

            lax.fori_loop(0, n_loop // NBUF, body, 0)
            for G in range(full_lo + n_loop, g_hi - GDEPTH):
                phase(G, G % NBUF, True, True)
            for G in range(g_hi - GDEPTH, g_hi):
                phase(G, G % NBUF, True, False)

        n_total = N_SLICES * n_per
        for G in range(n_total - SDEPTH, n_total):
            wait_store(G, G % NBUF)

    return emb(flat_ids, table)


def kernel(position_ids, embedding_weight):
    B0, S = position_ids.shape
    flat = position_ids.reshape(B0 * S).astype(jnp.int32)
    out = _gather_flat(flat, embedding_weight)
    return out.reshape(B0, S, HIDDEN)


# speedup vs baseline: 1.0437x; 1.0437x over previous
import functools

import jax
import jax.numpy as jnp
from jax import lax
from jax.experimental import pallas as pl
from jax.experimental.pallas import tpu as pltpu
from jax.experimental.pallas import tpu_sc as plsc

HIDDEN = 1024
NUM_CORES = 2
NUM_SUBCORES = 16
COLS = 128
N_SLICES = HIDDEN // (NUM_CORES * COLS)
CHUNK = 64
NBUF = 7
GDEPTH = 3
SDEPTH = NBUF - GDEPTH


def _gather_flat(flat_ids, table):
    B = flat_ids.shape[0]
    V = table.shape[0]
    b_per_w = B // NUM_SUBCORES
    n_per = b_per_w // CHUNK
    rows_per_sub = V // NUM_SUBCORES

    mesh = plsc.VectorSubcoreMesh(core_axis_name="c", subcore_axis_name="s")

    @functools.partial(
        pl.kernel,
        mesh=mesh,
        out_type=jax.ShapeDtypeStruct((B, HIDDEN), jnp.float32),
        scratch_types=[
            pltpu.VMEM_SHARED((V, COLS), jnp.float32),
            pltpu.VMEM((b_per_w,), jnp.int32),
        ]
        + [pltpu.VMEM((CHUNK, COLS), jnp.float32)] * NBUF
        + [pltpu.SemaphoreType.DMA] * NBUF,
    )
    def emb(idx_hbm, table_hbm, out_hbm, shared, idx_v, *rest):
        bufs = rest[:NBUF]
        sems = rest[NBUF:]
        c = lax.axis_index("c")
        s = lax.axis_index("s")
        base = s * b_per_w
        pltpu.sync_copy(idx_hbm.at[pl.ds(base, b_per_w)], idx_v)

        def row_off(G):
            return (G % n_per) * CHUNK

        def col0(G):
            return (c * N_SLICES + G // n_per) * COLS

        def start_gather(G, k):
            pltpu.async_copy(
                shared.at[idx_v.at[pl.ds(row_off(G), CHUNK)]], bufs[k], sems[k]
            )

        def wait_gather(G, k):
            pltpu.make_async_copy(
                shared.at[idx_v.at[pl.ds(row_off(G), CHUNK)]], bufs[k], sems[k]
            ).wait()

        def start_store(G, k):
            pltpu.async_copy(
                bufs[k],
                out_hbm.at[pl.ds(base + row_off(G), CHUNK), pl.ds(col0(G), COLS)],
                sems[k],
            )

        def wait_store(G, k):
            pltpu.make_async_copy(
                bufs[k],
                out_hbm.at[pl.ds(base + row_off(G), CHUNK), pl.ds(col0(G), COLS)],
                sems[k],
            ).wait()

        def phase(G, k, store_wait, gather_start):
            wait_gather(G, k)
            if store_wait:
                wait_store(G - SDEPTH, (k + GDEPTH) % NBUF)
            if gather_start:
                start_gather(G + GDEPTH, (k + GDEPTH) % NBUF)
            start_store(G, k)

        for i in range(N_SLICES):
            g_lo = i * n_per
            g_hi = g_lo + n_per
            if i > 0:
                plsc.subcore_barrier()
            pltpu.sync_copy(
                table_hbm.at[
                    pl.ds(s * rows_per_sub, rows_per_sub),
                    pl.ds((c * N_SLICES + i) * COLS, COLS),
                ],
                shared.at[pl.ds(s * rows_per_sub, rows_per_sub)],
            )
            plsc.subcore_barrier()

            for G in range(g_lo, g_lo + GDEPTH):
                start_gather(G, G % NBUF)
            head = SDEPTH if i == 0 else 0
            for G in range(g_lo, g_lo + head):
                phase(G, G % NBUF, False, True)
            full_lo = g_lo + head
            n_full = (g_hi - GDEPTH) - full_lo
            n_loop = (n_full // NBUF) * NBUF

            def body(j, carry):
                for p in range(NBUF):
                    G = full_lo + j * NBUF + p
                    phase(G, (full_lo + p) % NBUF, True, True)
                return carry

            lax.fori_loop(0, n_loop // NBUF, body, 0)
            for G in range(full_lo + n_loop, g_hi - GDEPTH):
                phase(G, G % NBUF, True, True)
            for G in range(g_hi - GDEPTH, g_hi):
                phase(G, G % NBUF, True, False)

        n_total = N_SLICES * n_per
        for G in range(n_total - SDEPTH, n_total):
            wait_store(G, G % NBUF)

    return emb(flat_ids, table)


def kernel(position_ids, embedding_weight):
    B0, S = position_ids.shape
    flat = position_ids.reshape(B0 * S).astype(jnp.int32)
    out = _gather_flat(flat, embedding_weight)
    return out.reshape(B0, S, HIDDEN)
